# baseline (speedup 1.0000x reference)
import functools

import jax
import jax.numpy as jnp
from jax import lax
from jax.experimental import pallas as pl
from jax.experimental.pallas import tpu as pltpu
from jax.experimental.pallas import tpu_sc as plsc

FULL_DIM = 2048
N_ROWS = 8192
L = 16
NC = 2
NS = 16
NW = NC * NS
XROWS_PER_W = N_ROWS // NW
RB = 8
BLK = RB * FULL_DIM
N_BLKS = XROWS_PER_W // RB
N_PAIRS = N_BLKS // 2
GROUPS = FULL_DIM // L
MC = 8
MPC = GROUPS // MC


def _make_permute_kernel():
    mesh = plsc.VectorSubcoreMesh(core_axis_name="c", subcore_axis_name="s")

    @functools.partial(
        pl.kernel,
        mesh=mesh,
        out_type=jax.ShapeDtypeStruct((N_ROWS * FULL_DIM,), jnp.float32),
        compiler_params=pltpu.CompilerParams(needs_layout_passes=False),
        scratch_types=[
            pltpu.VMEM((FULL_DIM,), jnp.int32),
            pltpu.VMEM((BLK,), jnp.float32),
            pltpu.VMEM((BLK,), jnp.float32),
            pltpu.VMEM((BLK,), jnp.float32),
            pltpu.VMEM((BLK,), jnp.float32),
            pltpu.SemaphoreType.DMA,
            pltpu.SemaphoreType.DMA,
            pltpu.SemaphoreType.DMA,
            pltpu.SemaphoreType.DMA,
        ],
    )
    def permute_rows(x_hbm, perm_hbm, out_hbm, perm_v,
                     in_a, in_b, out_a, out_b,
                     isem_a, isem_b, osem_a, osem_b):
        wid = lax.axis_index("s") * NC + lax.axis_index("c")
        w_base = wid * XROWS_PER_W * FULL_DIM

        pltpu.sync_copy(perm_hbm, perm_v)

        def permute_block(src, dst):
            for mc in range(MC):
                pv0 = [perm_v[pl.ds((mc * MPC + m) * L, L)]
                       for m in range(MPC)]

                @plsc.parallel_loop(0, RB, carry=pv0, unroll=2)
                def _row_body(r, pvs):
                    base = r * FULL_DIM
                    for m in range(MPC):
                        dst[pl.ds(base + (mc * MPC + m) * L, L)] = (
                            plsc.load_gather(src, [pvs[m]])
                        )
                    return [pv + FULL_DIM for pv in pvs]

        def pair_body(i, carry):
            o_a = w_base + (2 * i) * BLK
            o_b = o_a + BLK
            pltpu.async_copy(x_hbm.at[pl.ds(o_b, BLK)], in_b, isem_b)
            pltpu.make_async_copy(x_hbm.at[pl.ds(o_a, BLK)], in_a, isem_a).wait()

            @pl.when(i > 0)
            def _():
                pltpu.make_async_copy(
                    out_a, out_hbm.at[pl.ds(o_a, BLK)], osem_a).wait()

            permute_block(in_a, out_a)
            pltpu.async_copy(out_a, out_hbm.at[pl.ds(o_a, BLK)], osem_a)

            @pl.when(i < N_PAIRS - 1)
            def _():
                pltpu.async_copy(
                    x_hbm.at[pl.ds(o_b + BLK, BLK)], in_a, isem_a)

            pltpu.make_async_copy(x_hbm.at[pl.ds(o_b, BLK)], in_b, isem_b).wait()

            @pl.when(i > 0)
            def _():
                pltpu.make_async_copy(
                    out_b, out_hbm.at[pl.ds(o_b, BLK)], osem_b).wait()

            permute_block(in_b, out_b)
            pltpu.async_copy(out_b, out_hbm.at[pl.ds(o_b, BLK)], osem_b)
            return carry

        pltpu.async_copy(x_hbm.at[pl.ds(w_base, BLK)], in_a, isem_a)
        lax.fori_loop(0, N_PAIRS, pair_body, 0)
        pltpu.make_async_copy(out_a, out_hbm.at[pl.ds(w_base, BLK)], osem_a).wait()
        pltpu.make_async_copy(out_b, out_hbm.at[pl.ds(w_base, BLK)], osem_b).wait()

    return permute_rows


_PERMUTE_ROWS = _make_permute_kernel()


def kernel(x, permute):
    flat = jnp.reshape(x, (N_ROWS * FULL_DIM,))
    out = _PERMUTE_ROWS(flat, permute)
    return jnp.reshape(out, (N_ROWS, FULL_DIM))


# speedup vs baseline: 1.0079x; 1.0079x over previous
import functools

import jax
import jax.numpy as jnp
from jax import lax
from jax.experimental import pallas as pl
from jax.experimental.pallas import tpu as pltpu
from jax.experimental.pallas import tpu_sc as plsc

FULL_DIM = 2048
N_ROWS = 8192
L = 16
NC = 2
NS = 16
NW = NC * NS
XROWS_PER_W = N_ROWS // NW
RB = 8
BLK = RB * FULL_DIM
N_BLKS = XROWS_PER_W // RB
N_PAIRS = N_BLKS // 2
GROUPS = FULL_DIM // L
MC = 8
MPC = GROUPS // MC


def _make_permute_kernel():
    mesh = plsc.VectorSubcoreMesh(core_axis_name="c", subcore_axis_name="s")

    @functools.partial(
        pl.kernel,
        mesh=mesh,
        out_type=jax.ShapeDtypeStruct((N_ROWS * FULL_DIM,), jnp.float32),
        compiler_params=pltpu.CompilerParams(needs_layout_passes=False),
        scratch_types=[
            pltpu.VMEM((FULL_DIM,), jnp.int32),
            pltpu.VMEM((BLK,), jnp.float32),
            pltpu.VMEM((BLK,), jnp.float32),
            pltpu.VMEM((BLK,), jnp.float32),
            pltpu.VMEM((BLK,), jnp.float32),
            pltpu.SemaphoreType.DMA,
            pltpu.SemaphoreType.DMA,
            pltpu.SemaphoreType.DMA,
            pltpu.SemaphoreType.DMA,
        ],
    )
    def permute_rows(x_hbm, perm_hbm, out_hbm, perm_v,
                     in_a, in_b, out_a, out_b,
                     isem_a, isem_b, osem_a, osem_b):
        wid = lax.axis_index("s") * NC + lax.axis_index("c")
        w_base = wid * XROWS_PER_W * FULL_DIM

        pltpu.sync_copy(perm_hbm, perm_v)

        def permute_block(src, dst):
            for mc in range(MC):
                pv0 = [perm_v[pl.ds((mc * MPC + m) * L, L)]
                       for m in range(MPC)]

                @plsc.parallel_loop(0, RB, carry=pv0)
                def _row_body(r, pvs):
                    base = r * FULL_DIM
                    for m in range(MPC):
                        dst[pl.ds(base + (mc * MPC + m) * L, L)] = (
                            plsc.load_gather(src, [pvs[m]])
                        )
                    return [pv + FULL_DIM for pv in pvs]

        def pair_body(i, carry):
            o_a = w_base + (2 * i) * BLK
            o_b = o_a + BLK
            pltpu.async_copy(x_hbm.at[pl.ds(o_b, BLK)], in_b, isem_b)
            pltpu.make_async_copy(x_hbm.at[pl.ds(o_a, BLK)], in_a, isem_a).wait()

            @pl.when(i > 0)
            def _():
                pltpu.make_async_copy(
                    out_a, out_hbm.at[pl.ds(o_a, BLK)], osem_a).wait()

            permute_block(in_a, out_a)
            pltpu.async_copy(out_a, out_hbm.at[pl.ds(o_a, BLK)], osem_a)

            @pl.when(i < N_PAIRS - 1)
            def _():
                pltpu.async_copy(
                    x_hbm.at[pl.ds(o_b + BLK, BLK)], in_a, isem_a)

            pltpu.make_async_copy(x_hbm.at[pl.ds(o_b, BLK)], in_b, isem_b).wait()

            @pl.when(i > 0)
            def _():
                pltpu.make_async_copy(
                    out_b, out_hbm.at[pl.ds(o_b, BLK)], osem_b).wait()

            permute_block(in_b, out_b)
            pltpu.async_copy(out_b, out_hbm.at[pl.ds(o_b, BLK)], osem_b)
            return carry

        pltpu.async_copy(x_hbm.at[pl.ds(w_base, BLK)], in_a, isem_a)
        lax.fori_loop(0, N_PAIRS, pair_body, 0)
        pltpu.make_async_copy(out_a, out_hbm.at[pl.ds(w_base, BLK)], osem_a).wait()
        pltpu.make_async_copy(out_b, out_hbm.at[pl.ds(w_base, BLK)], osem_b).wait()

    return permute_rows


_PERMUTE_ROWS = _make_permute_kernel()


def kernel(x, permute):
    flat = jnp.reshape(x, (N_ROWS * FULL_DIM,))
    out = _PERMUTE_ROWS(flat, permute)
    return jnp.reshape(out, (N_ROWS, FULL_DIM))


# speedup vs baseline: 1.2125x; 1.2030x over previous
import functools

import jax
import jax.numpy as jnp
from jax import lax
from jax.experimental import pallas as pl
from jax.experimental.pallas import tpu as pltpu
from jax.experimental.pallas import tpu_sc as plsc

FULL_DIM = 2048
N_ROWS = 8192
L = 16
NC = 2
NS = 16
NW = NC * NS
XROWS_PER_W = N_ROWS // NW
RB = 8
BLK = RB * FULL_DIM
N_BLKS = XROWS_PER_W // RB
GROUPS = FULL_DIM // L


def _make_permute_kernel():
    mesh = plsc.VectorSubcoreMesh(core_axis_name="c", subcore_axis_name="s")

    @functools.partial(
        pl.kernel,
        mesh=mesh,
        out_type=jax.ShapeDtypeStruct((N_ROWS * FULL_DIM,), jnp.float32),
        compiler_params=pltpu.CompilerParams(needs_layout_passes=False),
        scratch_types=[
            pltpu.VMEM((FULL_DIM,), jnp.int32),
            pltpu.VMEM((2 * BLK,), jnp.float32),
            pltpu.VMEM((2 * BLK,), jnp.float32),
            pltpu.SemaphoreType.DMA((2,)),
            pltpu.SemaphoreType.DMA((2,)),
        ],
    )
    def permute_rows(x_hbm, perm_hbm, out_hbm, perm_v, in2, out2, isem, osem):
        wid = lax.axis_index("s") * NC + lax.axis_index("c")
        w_base = wid * XROWS_PER_W * FULL_DIM

        pltpu.sync_copy(perm_hbm, perm_v)

        def blk_body(b, carry):
            p = b & 1
            q = 1 - p
            hbm_off = w_base + b * BLK

            @pl.when(b + 1 < N_BLKS)
            def _():
                pltpu.async_copy(
                    x_hbm.at[pl.ds(hbm_off + BLK, BLK)],
                    in2.at[pl.ds(q * BLK, BLK)], isem.at[q])

            pltpu.make_async_copy(
                x_hbm.at[pl.ds(hbm_off, BLK)],
                in2.at[pl.ds(p * BLK, BLK)], isem.at[p]).wait()

            @pl.when(b >= 2)
            def _():
                pltpu.make_async_copy(
                    out2.at[pl.ds(p * BLK, BLK)],
                    out_hbm.at[pl.ds(hbm_off, BLK)], osem.at[p]).wait()

            sbase = [p * BLK + r * FULL_DIM for r in range(RB)]
            rbase = [jnp.full((L,), 0, jnp.int32) + sb for sb in sbase]

            @plsc.parallel_loop(0, GROUPS)
            def _group(m):
                pvec = perm_v[pl.ds(m * L, L)]
                o = m * L
                for r in range(RB):
                    out2[pl.ds(sbase[r] + o, L)] = plsc.load_gather(
                        in2, [pvec + rbase[r]])

            pltpu.async_copy(
                out2.at[pl.ds(p * BLK, BLK)],
                out_hbm.at[pl.ds(hbm_off, BLK)], osem.at[p])
            return carry

        pltpu.async_copy(
            x_hbm.at[pl.ds(w_base, BLK)], in2.at[pl.ds(0, BLK)], isem.at[0])
        lax.fori_loop(0, N_BLKS, blk_body, 0)
        for p in range(2):
            pltpu.make_async_copy(
                out2.at[pl.ds(p * BLK, BLK)],
                out_hbm.at[pl.ds(w_base, BLK)], osem.at[p]).wait()

    return permute_rows


_PERMUTE_ROWS = _make_permute_kernel()


def kernel(x, permute):
    flat = jnp.reshape(x, (N_ROWS * FULL_DIM,))
    out = _PERMUTE_ROWS(flat, permute)
    return jnp.reshape(out, (N_ROWS, FULL_DIM))
